# Initial kernel scaffold; baseline (speedup 1.0000x reference)
import jax
import jax.numpy as jnp
from jax.experimental import pallas as pl


def kernel(xyz1, xyz2, pts1, pts2, W1, b1, g1, bt1, W2, b2, g2, bt2):
    raise NotImplementedError("write your pallas kernel here")


# speedup vs baseline: 20.5408x; 20.5408x over previous
import functools

import jax
import jax.numpy as jnp
from jax.experimental import pallas as pl
from jax.experimental.pallas import tpu as pltpu
from jax.experimental.pallas import tpu_sc as plsc

_NB = 256


def _knn_body(x1_ref, x2t_ref, idx_ref, w_ref, *, S, NB):
    b = pl.program_id(0)
    x1 = x1_ref[0]
    x2 = x2t_ref[0]
    x1b = x1.astype(jnp.bfloat16).astype(jnp.float32)
    x2b = x2.astype(jnp.bfloat16).astype(jnp.float32)
    cross = (x1b[:, 0:1] * x2b[0:1, :]
             + x1b[:, 1:2] * x2b[1:2, :]
             + x1b[:, 2:3] * x2b[2:3, :])
    n2 = x2[0:1, :] ** 2 + x2[1:2, :] ** 2 + x2[2:3, :] ** 2
    n1 = x1[:, 0:1] ** 2 + x1[:, 1:2] ** 2 + x1[:, 2:3] ** 2
    s = (n1 - 2.0 * cross) + n2
    iota = jax.lax.broadcasted_iota(jnp.int32, (NB, S), 1)
    ds, idxs = [], []
    for k in range(3):
        m = jnp.min(s, axis=1, keepdims=True)
        idx = jnp.min(jnp.where(s <= m, iota, S), axis=1, keepdims=True)
        ds.append(m)
        idxs.append(idx)
        if k < 2:
            s = jnp.where(iota == idx, jnp.float32(jnp.inf), s)
    d3 = jnp.concatenate(ds, axis=1)
    w3 = 1.0 / (d3 + 1e-8)
    w3 = w3 / jnp.sum(w3, axis=1, keepdims=True)
    idx_ref[0] = jnp.concatenate(idxs, axis=1) + b * S
    w_ref[0] = w3


def _gather_rows(table, idx_flat):
    K = idx_flat.shape[1]
    C = table.shape[1]
    WIN = 128
    mesh = plsc.VectorSubcoreMesh(core_axis_name="c", subcore_axis_name="s")

    @pl.kernel(out_type=jax.ShapeDtypeStruct((K, C), table.dtype), mesh=mesh)
    def _k(tab_hbm, i_hbm, o_hbm):
        def body(i_vmem, o_vmem):
            pltpu.sync_copy(tab_hbm.at[i_vmem.at[0]], o_vmem)

        pltpu.emit_pipeline(
            body,
            grid=(K // WIN,),
            in_specs=[pl.BlockSpec((1, WIN), index_map=lambda i: (0, i))],
            out_specs=[pl.BlockSpec((WIN, C), index_map=lambda i: (i, 0))],
            core_axis_name=("c", "s"),
            dimension_semantics=(pltpu.PARALLEL,),
        )(i_hbm, o_hbm)

    return _k(table, idx_flat)


def _stats_update(st_ref, y, first):
    @pl.when(first)
    def _():
        st_ref[...] = jnp.zeros_like(st_ref)

    ps = jnp.sum(y, axis=0, keepdims=True)
    psq = jnp.sum(y * y, axis=0, keepdims=True)
    st_ref[0:1, :] = st_ref[0:1, :] + ps
    st_ref[1:2, :] = st_ref[1:2, :] + psq


def _mlp1_body(g_ref, w_ref, p1_ref, w1t_ref, b1_ref, y_ref, st_ref, *, C2):
    w = w_ref[...]
    interp = (g_ref[0, :, :C2] * w[:, 0:1]
              + g_ref[1, :, :C2] * w[:, 1:2]
              + g_ref[2, :, :C2] * w[:, 2:3])
    h = jnp.concatenate([p1_ref[...], interp], axis=1)
    y = jnp.dot(h, w1t_ref[...], preferred_element_type=jnp.float32,
                precision=jax.lax.Precision.HIGHEST) + b1_ref[0:1, :]
    y_ref[...] = y
    _stats_update(st_ref, y, pl.program_id(0) == 0)


def _mlp2_body(y1_ref, sc_ref, w2t_ref, b2_ref, y_ref, st_ref):
    z = jnp.maximum(y1_ref[...] * sc_ref[0:1, :] + sc_ref[1:2, :], 0.0)
    y = jnp.dot(z, w2t_ref[...], preferred_element_type=jnp.float32,
                precision=jax.lax.Precision.HIGHEST) + b2_ref[0:1, :]
    y_ref[...] = y
    _stats_update(st_ref, y, pl.program_id(0) == 0)


def _bn_relu_body(y_ref, sc_ref, o_ref):
    o_ref[...] = jnp.maximum(y_ref[...] * sc_ref[0:1, :] + sc_ref[1:2, :], 0.0)


def _affine(st, M, gamma, beta):
    mean = st[0] / M
    var = st[1] / M - mean * mean
    scale = gamma / jnp.sqrt(var + 1e-5)
    shift = beta - mean * scale
    return jnp.stack([scale, shift], axis=0)


def kernel(xyz1, xyz2, pts1, pts2, W1, b1, g1, bt1, W2, b2, g2, bt2):
    B, N, _ = xyz1.shape
    S = xyz2.shape[1]
    C1 = pts1.shape[2]
    C2 = pts2.shape[2]
    CO1 = W1.shape[0]
    CO2 = W2.shape[0]
    M = B * N
    NB = _NB

    knn = pl.pallas_call(
        functools.partial(_knn_body, S=S, NB=NB),
        grid=(B, N // NB),
        in_specs=[
            pl.BlockSpec((1, NB, 3), lambda b, i: (b, i, 0)),
            pl.BlockSpec((1, 3, S), lambda b, i: (b, 0, 0)),
        ],
        out_specs=[
            pl.BlockSpec((1, NB, 3), lambda b, i: (b, i, 0)),
            pl.BlockSpec((1, NB, 3), lambda b, i: (b, i, 0)),
        ],
        out_shape=[
            jax.ShapeDtypeStruct((B, N, 3), jnp.int32),
            jax.ShapeDtypeStruct((B, N, 3), jnp.float32),
        ],
    )
    xyz2t = jnp.transpose(xyz2, (0, 2, 1))
    idx, w = knn(xyz1, xyz2t)

    idx_t = jnp.transpose(idx.reshape(M, 3), (1, 0)).reshape(1, 3 * M)
    table = jnp.pad(pts2.reshape(B * S, C2), ((0, 0), (0, 128 - C2)))
    gathered = _gather_rows(table, idx_t)
    g3 = gathered.reshape(3, M, 128)

    mlp1 = pl.pallas_call(
        functools.partial(_mlp1_body, C2=C2),
        grid=(M // NB,),
        in_specs=[
            pl.BlockSpec((3, NB, 128), lambda i: (0, i, 0)),
            pl.BlockSpec((NB, 3), lambda i: (i, 0)),
            pl.BlockSpec((NB, C1), lambda i: (i, 0)),
            pl.BlockSpec((C1 + C2, CO1), lambda i: (0, 0)),
            pl.BlockSpec((1, CO1), lambda i: (0, 0)),
        ],
        out_specs=[
            pl.BlockSpec((NB, CO1), lambda i: (i, 0)),
            pl.BlockSpec((8, CO1), lambda i: (0, 0)),
        ],
        out_shape=[
            jax.ShapeDtypeStruct((M, CO1), jnp.float32),
            jax.ShapeDtypeStruct((8, CO1), jnp.float32),
        ],
    )
    y1, st1 = mlp1(g3, w.reshape(M, 3), pts1.reshape(M, C1),
                   jnp.transpose(W1, (1, 0)), b1.reshape(1, CO1))
    sc1 = _affine(st1, M, g1, bt1)

    mlp2 = pl.pallas_call(
        _mlp2_body,
        grid=(M // NB,),
        in_specs=[
            pl.BlockSpec((NB, CO1), lambda i: (i, 0)),
            pl.BlockSpec((2, CO1), lambda i: (0, 0)),
            pl.BlockSpec((CO1, CO2), lambda i: (0, 0)),
            pl.BlockSpec((1, CO2), lambda i: (0, 0)),
        ],
        out_specs=[
            pl.BlockSpec((NB, CO2), lambda i: (i, 0)),
            pl.BlockSpec((8, CO2), lambda i: (0, 0)),
        ],
        out_shape=[
            jax.ShapeDtypeStruct((M, CO2), jnp.float32),
            jax.ShapeDtypeStruct((8, CO2), jnp.float32),
        ],
    )
    y2, st2 = mlp2(y1, sc1, jnp.transpose(W2, (1, 0)), b2.reshape(1, CO2))
    sc2 = _affine(st2, M, g2, bt2)

    bn_relu = pl.pallas_call(
        _bn_relu_body,
        grid=(M // NB,),
        in_specs=[
            pl.BlockSpec((NB, CO2), lambda i: (i, 0)),
            pl.BlockSpec((2, CO2), lambda i: (0, 0)),
        ],
        out_specs=pl.BlockSpec((NB, CO2), lambda i: (i, 0)),
        out_shape=jax.ShapeDtypeStruct((M, CO2), jnp.float32),
    )
    out = bn_relu(y2, sc2)
    return out.reshape(B, N, CO2)
